# baseline (speedup 1.0000x reference)
import jax
import jax.numpy as jnp
from jax import lax
from jax.experimental import pallas as pl
from jax.experimental.pallas import tpu as pltpu
from jax.experimental.pallas import tpu_sc as plsc

N = 10000
E = 320000
R = 7
D = 128

NC = 2
NS = 16
NW = NC * NS
EPW = E // NW
CH = 64
NCH = 159
EPWP = NCH * CH
NBUF = 3
RSEG = EPWP // 4
RPW = 632
RLAST = N - 15 * RPW
L = 16


def _sc_body(y_hbm, src_hbm, rel_hbm, dst_hbm, w_hbm, z_hbm,
             out0_hbm, out1_hbm,
             srcv, relv, dstv, w0, w1, w2, rows0, rows1, rows2, acc,
             g0, g1, g2, s0, s1, s2, t0, t1, t2):
    c = lax.axis_index("c")
    s = lax.axis_index("s")
    wid = c * NS + s
    rows = (rows0, rows1, rows2)
    wsm = (w0, w1, w2)
    gsem = (g0, g1, g2)
    ssem = (s0, s1, s2)
    wsem = (t0, t1, t2)

    @pl.when(s < 15)
    def _():
        pltpu.sync_copy(z_hbm, acc.at[pl.ds(s * RPW, RPW)])

    @pl.when(s == 15)
    def _():
        pltpu.sync_copy(z_hbm.at[pl.ds(0, RLAST)],
                        acc.at[pl.ds(15 * RPW, RLAST)])

    pltpu.sync_copy(src_hbm.at[wid], srcv)
    pltpu.sync_copy(dst_hbm.at[wid], dstv)

    for seg in range(EPWP // RSEG):
        pltpu.sync_copy(rel_hbm.at[wid].at[pl.ds(seg * RSEG, RSEG)], relv)

        def _gidx(t):
            gl = pl.ds(seg * RSEG + t * L, L)
            srcv[gl] = relv[pl.ds(t * L, L)] * N + srcv[gl]
        plsc.parallel_loop(0, RSEG // L, 1, unroll=4)(_gidx)

    plsc.subcore_barrier()

    def _gather(ci, b):
        pltpu.async_copy(y_hbm.at[srcv.at[pl.ds(ci * CH, CH)]],
                         rows[b], gsem[b])
        pltpu.async_copy(w_hbm.at[wid].at[pl.ds(ci * CH, CH)],
                         wsm[b], wsem[b])

    def _gather_wait(ci, b):
        pltpu.make_async_copy(y_hbm.at[srcv.at[pl.ds(ci * CH, CH)]],
                              rows[b], gsem[b]).wait()
        pltpu.make_async_copy(w_hbm.at[wid].at[pl.ds(ci * CH, CH)],
                              wsm[b], wsem[b]).wait()

    def _scatter_wait(ci, b):
        pltpu.make_async_copy(rows[b], acc.at[dstv.at[ci]], ssem[b]).wait()

    _gather(0, 0)
    _gather(1, 1)

    def _round(i, _):
        for b in range(NBUF):
            ci = i * NBUF + b
            _gather_wait(ci, b)

            b2 = (b + 2) % NBUF

            @pl.when(ci >= 1)
            def _():
                _scatter_wait(ci - 1, b2)

            @pl.when(ci + 2 < NCH)
            def _():
                _gather(ci + 2, b2)

            def _scale(k):
                wbc = plsc.load_gather(
                    wsm[b], [jnp.full((L,), k, jnp.int32)])
                for db in range(D // L):
                    sl = pl.ds(db * L, L)
                    rows[b][k, sl] = rows[b][k, sl] * wbc
            plsc.parallel_loop(0, CH, 1, unroll=8)(_scale)

            pltpu.async_copy(rows[b], acc.at[dstv.at[ci]], ssem[b],
                             add=True)
        return 0
    lax.fori_loop(0, NCH // NBUF, _round, 0)

    _scatter_wait(NCH - 1, (NCH - 1) % NBUF)

    plsc.subcore_barrier()

    def _writeout(out_hbm):
        @pl.when(s < 15)
        def _():
            rsl = pl.ds(s * RPW, RPW)
            pltpu.sync_copy(acc.at[rsl], out_hbm.at[rsl])

        @pl.when(s == 15)
        def _():
            rsl = pl.ds(15 * RPW, RLAST)
            pltpu.sync_copy(acc.at[rsl], out_hbm.at[rsl])

    @pl.when(c == 0)
    def _():
        _writeout(out0_hbm)

    @pl.when(c == 1)
    def _():
        _writeout(out1_hbm)


def _sc_scatter(y2, src2, rel2, dst3, w2, zrows):
    mesh = plsc.VectorSubcoreMesh(core_axis_name="c", subcore_axis_name="s")
    f = pl.kernel(
        _sc_body,
        out_type=(jax.ShapeDtypeStruct((N, D), jnp.float32),
                  jax.ShapeDtypeStruct((N, D), jnp.float32)),
        mesh=mesh,
        compiler_params=pltpu.CompilerParams(
            needs_layout_passes=False,
            use_tc_tiling_on_sc=False,
        ),
        scratch_types=(
            pltpu.VMEM((EPWP,), jnp.int32),
            pltpu.VMEM((RSEG,), jnp.int32),
            pltpu.VMEM((NCH, CH), jnp.int32),
            pltpu.VMEM((CH,), jnp.float32),
            pltpu.VMEM((CH,), jnp.float32),
            pltpu.VMEM((CH,), jnp.float32),
            pltpu.VMEM((CH, D), jnp.float32),
            pltpu.VMEM((CH, D), jnp.float32),
            pltpu.VMEM((CH, D), jnp.float32),
            pltpu.VMEM_SHARED((N, D), jnp.float32),
            pltpu.SemaphoreType.DMA,
            pltpu.SemaphoreType.DMA,
            pltpu.SemaphoreType.DMA,
            pltpu.SemaphoreType.DMA,
            pltpu.SemaphoreType.DMA,
            pltpu.SemaphoreType.DMA,
            pltpu.SemaphoreType.DMA,
            pltpu.SemaphoreType.DMA,
            pltpu.SemaphoreType.DMA,
        ),
    )
    return f(y2, src2, rel2, dst3, w2, zrows)


def _proj_body(h_ref, m_ref, wsT_ref, b_ref, y_ref, s_ref):
    h = h_ref[...]
    y_ref[0] = jnp.dot(h, m_ref[0], preferred_element_type=jnp.float32)

    @pl.when(pl.program_id(1) == 0)
    def _():
        s_ref[...] = (jnp.dot(h, wsT_ref[...],
                              preferred_element_type=jnp.float32)
                      + b_ref[...])


def _proj(h, m3, wsT, b):
    bm = 1000
    return pl.pallas_call(
        _proj_body,
        grid=(N // bm, R),
        in_specs=[pl.BlockSpec((bm, D), lambda i, r: (i, 0)),
                  pl.BlockSpec((1, D, D), lambda i, r: (r, 0, 0)),
                  pl.BlockSpec((D, D), lambda i, r: (0, 0)),
                  pl.BlockSpec((1, D), lambda i, r: (0, 0))],
        out_specs=[pl.BlockSpec((1, bm, D), lambda i, r: (r, i, 0)),
                   pl.BlockSpec((bm, D), lambda i, r: (i, 0))],
        out_shape=[jax.ShapeDtypeStruct((R, N, D), jnp.float32),
                   jax.ShapeDtypeStruct((N, D), jnp.float32)],
    )(h, m3, wsT, b)


def _comb_body(p0_ref, p1_ref, s_ref, o_ref):
    o_ref[...] = jnp.maximum(p0_ref[...] + p1_ref[...] + s_ref[...], 0.0)


def _combine(p0, p1, s):
    bm = 1000
    spec = pl.BlockSpec((bm, D), lambda i: (i, 0))
    return pl.pallas_call(
        _comb_body,
        grid=(N // bm,),
        in_specs=[spec, spec, spec],
        out_specs=spec,
        out_shape=jax.ShapeDtypeStruct((N, D), jnp.float32),
    )(p0, p1, s)


def kernel(x, edge_index, edge_relation, edge_weight,
           W_lin0, b_lin0, W_self0, b_self0,
           W_lin1, b_lin1, W_self1, b_self1):
    pad = ((0, 0), (0, EPWP - EPW))
    src2 = jnp.pad(edge_index[0].astype(jnp.int32).reshape(NW, EPW), pad)
    rel2 = jnp.pad(edge_relation.astype(jnp.int32).reshape(NW, EPW), pad)
    dst3 = jnp.pad(edge_index[1].astype(jnp.int32).reshape(NW, EPW),
                   pad).reshape(NW, NCH, CH)
    w2 = jnp.pad(edge_weight.astype(jnp.float32).reshape(NW, EPW), pad)
    zrows = jnp.zeros((RPW, D), jnp.float32)

    def mk_m(W_lin):
        return jnp.transpose(W_lin.reshape(D, R, D), (1, 2, 0))

    m0, m1 = mk_m(W_lin0), mk_m(W_lin1)
    b0 = (b_lin0 + b_self0).reshape(1, D)
    b1 = (b_lin1 + b_self1).reshape(1, D)

    y0, s0 = _proj(x, m0, W_self0.T, b0)
    p00, p01 = _sc_scatter(y0.reshape(R * N, D), src2, rel2, dst3, w2, zrows)
    h = _combine(p00, p01, s0)

    y1, s1 = _proj(h, m1, W_self1.T, b1)
    p10, p11 = _sc_scatter(y1.reshape(R * N, D), src2, rel2, dst3, w2, zrows)
    return _combine(p10, p11, s1)
